# Initial kernel scaffold; baseline (speedup 1.0000x reference)
import jax
import jax.numpy as jnp
from jax.experimental import pallas as pl


def kernel(heads, rels, tails, dates, ent_embs_h, ent_embs_t, rel_embs_f, rel_embs_i, tim_embs_f):
    raise NotImplementedError("write your pallas kernel here")


# speedup vs baseline: 3.4263x; 3.4263x over previous
import functools

import jax
import jax.numpy as jnp
from jax import lax
from jax.experimental import pallas as pl
from jax.experimental.pallas import tpu as pltpu
from jax.experimental.pallas import tpu_sc as plsc

_B = 16384
_D = 128
_TD = 64
_NTIME = 365
_NC = 2
_NS = 16
_NW = _NC * _NS
_BPW = _B // _NW
_CHUNK = 64
_NCHUNK = _BPW // _CHUNK

_mesh = plsc.VectorSubcoreMesh(core_axis_name="c", subcore_axis_name="s")


@functools.partial(
    pl.kernel,
    mesh=_mesh,
    compiler_params=pltpu.CompilerParams(needs_layout_passes=False),
    out_type=jax.ShapeDtypeStruct((_B,), jnp.float32),
    scratch_types=[
        pltpu.VMEM((_BPW,), jnp.int32),
        pltpu.VMEM((_BPW,), jnp.int32),
        pltpu.VMEM((_BPW,), jnp.int32),
        pltpu.VMEM((_BPW,), jnp.int32),
        pltpu.VMEM((_CHUNK, _D), jnp.float32),
        pltpu.VMEM((_CHUNK, _D), jnp.float32),
        pltpu.VMEM((_CHUNK, _D), jnp.float32),
        pltpu.VMEM((_CHUNK, _D), jnp.float32),
        pltpu.VMEM((_CHUNK, _D), jnp.float32),
        pltpu.VMEM((_CHUNK, _D), jnp.float32),
        pltpu.VMEM((_CHUNK, _D), jnp.float32),
        pltpu.VMEM((_BPW,), jnp.float32),
        pltpu.SemaphoreType.DMA,
    ],
)
def _tsimple_sc(heads_hbm, rels_hbm, tails_hbm, dates_hbm,
                eh_hbm, et_hbm, rf_hbm, ri_hbm, tf_hbm,
                out_hbm,
                idx_h, idx_r, idx_t, idx_d,
                h1, r1, t1, h2, r2, t2, tv, outb, sem):
    wid = lax.axis_index("s") * _NC + lax.axis_index("c")
    base = wid * _BPW

    pltpu.sync_copy(heads_hbm.at[pl.ds(base, _BPW)], idx_h)
    pltpu.sync_copy(rels_hbm.at[pl.ds(base, _BPW)], idx_r)
    pltpu.sync_copy(tails_hbm.at[pl.ds(base, _BPW)], idx_t)
    pltpu.sync_copy(dates_hbm.at[pl.ds(base, _BPW)], idx_d)

    lane_iota = lax.iota(jnp.int32, 16)
    idx15 = jnp.full((16, 1), 15, jnp.int32)
    gdn = lax.GatherDimensionNumbers(
        offset_dims=(), collapsed_slice_dims=(0,), start_index_map=(0,))

    def chunk_body(ci, carry):
        sl_c = pl.ds(ci * _CHUNK, _CHUNK)
        cps = [
            pltpu.async_copy(eh_hbm.at[idx_h.at[sl_c]], h1, sem),
            pltpu.async_copy(rf_hbm.at[idx_r.at[sl_c]], r1, sem),
            pltpu.async_copy(et_hbm.at[idx_t.at[sl_c]], t1, sem),
            pltpu.async_copy(eh_hbm.at[idx_t.at[sl_c]], h2, sem),
            pltpu.async_copy(ri_hbm.at[idx_r.at[sl_c]], r2, sem),
            pltpu.async_copy(et_hbm.at[idx_h.at[sl_c]], t2, sem),
            pltpu.async_copy(tf_hbm.at[idx_d.at[sl_c]], tv, sem),
        ]
        for cp in cps:
            cp.wait()

        for g in range(_CHUNK // 16):
            def e_body(l, res):
                e = g * 16 + l
                acc = jnp.zeros((16,), jnp.float32)
                for k in range(_D // 16):
                    sl = pl.ds(k * 16, 16)
                    term = h1[e, sl] * r1[e, sl] * t1[e, sl] \
                        + h2[e, sl] * r2[e, sl] * t2[e, sl]
                    if k < _TD // 16:
                        term = term * tv[e, sl]
                    acc = acc + term
                csum = jnp.cumsum(acc)
                total = lax.gather(
                    csum, idx15, dimension_numbers=gdn, slice_sizes=(1,),
                    mode=lax.GatherScatterMode.PROMISE_IN_BOUNDS)
                return jnp.where(lane_iota == l, total, res)

            res = lax.fori_loop(0, 16, e_body, jnp.zeros((16,), jnp.float32))
            outb[pl.ds(ci * _CHUNK + g * 16, 16)] = res * 0.5
        return carry

    lax.fori_loop(0, _NCHUNK, chunk_body, 0)
    pltpu.sync_copy(outb, out_hbm.at[pl.ds(base, _BPW)])


def kernel(heads, rels, tails, dates, ent_embs_h, ent_embs_t,
           rel_embs_f, rel_embs_i, tim_embs_f):
    tf_pad = jnp.pad(tim_embs_f, ((0, 0), (0, _D - _TD)))
    return _tsimple_sc(heads, rels, tails, dates, ent_embs_h, ent_embs_t,
                       rel_embs_f, rel_embs_i, tf_pad)


# speedup vs baseline: 4.2741x; 1.2474x over previous
import functools

import jax
import jax.numpy as jnp
from jax import lax
from jax.experimental import pallas as pl
from jax.experimental.pallas import tpu as pltpu
from jax.experimental.pallas import tpu_sc as plsc

_B = 16384
_D = 128
_TD = 64
_NC = 2
_NS = 16
_NW = _NC * _NS
_BPW = _B // _NW
_CHUNK = 64
_NCHUNK = _BPW // _CHUNK

_mesh = plsc.VectorSubcoreMesh(core_axis_name="c", subcore_axis_name="s")

_row_buf = pltpu.VMEM((_CHUNK, _D), jnp.float32)


@functools.partial(
    pl.kernel,
    mesh=_mesh,
    compiler_params=pltpu.CompilerParams(needs_layout_passes=False),
    out_type=jax.ShapeDtypeStruct((_B,), jnp.float32),
    scratch_types=[
        pltpu.VMEM((_BPW,), jnp.int32),
        pltpu.VMEM((_BPW,), jnp.int32),
        pltpu.VMEM((_BPW,), jnp.int32),
        pltpu.VMEM((_BPW,), jnp.int32),
        [_row_buf] * 7,
        [_row_buf] * 7,
        pltpu.VMEM((_BPW,), jnp.float32),
        pltpu.SemaphoreType.DMA,
        pltpu.SemaphoreType.DMA,
    ],
)
def _tsimple_sc(heads_hbm, rels_hbm, tails_hbm, dates_hbm,
                eh_hbm, et_hbm, rf_hbm, ri_hbm, tf_hbm,
                out_hbm,
                idx_h, idx_r, idx_t, idx_d,
                bufs_a, bufs_b, outb, sem_a, sem_b):
    wid = lax.axis_index("s") * _NC + lax.axis_index("c")
    base = wid * _BPW

    pltpu.sync_copy(heads_hbm.at[pl.ds(base, _BPW)], idx_h)
    pltpu.sync_copy(rels_hbm.at[pl.ds(base, _BPW)], idx_r)
    pltpu.sync_copy(tails_hbm.at[pl.ds(base, _BPW)], idx_t)
    pltpu.sync_copy(dates_hbm.at[pl.ds(base, _BPW)], idx_d)

    def start_set(ci, bufs, sem):
        sl_c = pl.ds(ci * _CHUNK, _CHUNK)
        h1, r1, t1, h2, r2, t2, tv = bufs
        pltpu.async_copy(eh_hbm.at[idx_h.at[sl_c]], h1, sem)
        pltpu.async_copy(rf_hbm.at[idx_r.at[sl_c]], r1, sem)
        pltpu.async_copy(et_hbm.at[idx_t.at[sl_c]], t1, sem)
        pltpu.async_copy(eh_hbm.at[idx_t.at[sl_c]], h2, sem)
        pltpu.async_copy(ri_hbm.at[idx_r.at[sl_c]], r2, sem)
        pltpu.async_copy(et_hbm.at[idx_h.at[sl_c]], t2, sem)
        pltpu.async_copy(tf_hbm.at[idx_d.at[sl_c]], tv, sem)

    def wait_set(bufs, sem):
        for b in bufs:
            pltpu.make_async_copy(eh_hbm.at[pl.ds(0, _CHUNK)], b, sem).wait()

    lane_iota = lax.iota(jnp.int32, 16)
    idx15 = jnp.full((16, 1), 15, jnp.int32)
    gdn = lax.GatherDimensionNumbers(
        offset_dims=(), collapsed_slice_dims=(0,), start_index_map=(0,))

    def compute_set(ci, bufs):
        h1, r1, t1, h2, r2, t2, tv = bufs
        for g in range(_CHUNK // 16):
            def e_body(l, res):
                e = g * 16 + l
                acc = jnp.zeros((16,), jnp.float32)
                for k in range(_D // 16):
                    sl = pl.ds(k * 16, 16)
                    term = h1[e, sl] * r1[e, sl] * t1[e, sl] \
                        + h2[e, sl] * r2[e, sl] * t2[e, sl]
                    if k < _TD // 16:
                        term = term * tv[e, sl]
                    acc = acc + term
                csum = jnp.cumsum(acc)
                total = lax.gather(
                    csum, idx15, dimension_numbers=gdn, slice_sizes=(1,),
                    mode=lax.GatherScatterMode.PROMISE_IN_BOUNDS)
                return jnp.where(lane_iota == l, total, res)

            res = lax.fori_loop(0, 16, e_body, jnp.zeros((16,), jnp.float32))
            outb[pl.ds(ci * _CHUNK + g * 16, 16)] = res * 0.5

    n2 = _NCHUNK // 2
    start_set(0, bufs_a, sem_a)

    def pair_body(cj, carry):
        ci0 = 2 * cj
        start_set(ci0 + 1, bufs_b, sem_b)
        wait_set(bufs_a, sem_a)
        compute_set(ci0, bufs_a)

        @pl.when(cj < n2 - 1)
        def _():
            start_set(ci0 + 2, bufs_a, sem_a)

        wait_set(bufs_b, sem_b)
        compute_set(ci0 + 1, bufs_b)
        return carry

    lax.fori_loop(0, n2, pair_body, 0)
    pltpu.sync_copy(outb, out_hbm.at[pl.ds(base, _BPW)])


def kernel(heads, rels, tails, dates, ent_embs_h, ent_embs_t,
           rel_embs_f, rel_embs_i, tim_embs_f):
    tf_pad = jnp.pad(tim_embs_f, ((0, 0), (0, _D - _TD)))
    return _tsimple_sc(heads, rels, tails, dates, ent_embs_h, ent_embs_t,
                       rel_embs_f, rel_embs_i, tf_pad)
